# baseline (speedup 1.0000x reference)
import functools

import jax
import jax.numpy as jnp
from jax import lax
from jax.experimental import pallas as pl
from jax.experimental.pallas import tpu as pltpu
from jax.experimental.pallas import tpu_sc as plsc

NC = 2
NS = 16
NW = NC * NS
LANES = 16
CHUNK = 80


@functools.lru_cache(maxsize=None)
def _make_sc_agg(n_nodes, n_pad, d, n_chunks, with_deg):
    rows_per_tile = n_pad // NS
    assert rows_per_tile % CHUNK == 0
    nz = rows_per_tile // CHUNK

    mesh = plsc.VectorSubcoreMesh(core_axis_name="c", subcore_axis_name="s")

    out_type = [jax.ShapeDtypeStruct((NC, n_pad, d), jnp.float32)]
    if with_deg:
        out_type.append(jax.ShapeDtypeStruct((NC, n_pad), jnp.float32))

    assert n_chunks >= 5 and (n_chunks - 5) % 3 == 0

    scratch = [
        pltpu.VMEM((n_chunks, CHUNK), jnp.int32),
        [pltpu.VMEM((CHUNK,), jnp.int32)] * 3,
        [pltpu.VMEM((CHUNK,), jnp.int32)] * 3,
        [pltpu.VMEM((CHUNK, d), jnp.float32)] * 3,
        pltpu.VMEM((CHUNK,), jnp.float32),
        pltpu.VMEM((CHUNK,), jnp.float32),
        pltpu.VMEM_SHARED((n_pad, d), jnp.float32),
        pltpu.VMEM_SHARED((n_pad,), jnp.float32),
        [pltpu.SemaphoreType.DMA] * 3,
        [pltpu.SemaphoreType.DMA] * 3,
        pltpu.SemaphoreType.DMA,
    ]

    def body(table_hbm, pk_hbm, *refs):
        if with_deg:
            out_hbm, deg_hbm = refs[0], refs[1]
            rest = refs[2:]
        else:
            out_hbm = refs[0]
            rest = refs[1:]
        (pk_v, src_b, dst_b, rows_b, ones_v, zer_v,
         acc_sh, deg_sh, gsem, ssem, dsem) = rest

        cid = lax.axis_index("c")
        sid = lax.axis_index("s")
        wid = sid * NC + cid

        pltpu.sync_copy(pk_hbm.at[wid], pk_v)

        def unpack(c, j):
            for k in range(CHUNK // LANES):
                v = pk_v[c, pl.ds(k * LANES, LANES)]
                src_b[j][pl.ds(k * LANES, LANES)] = lax.shift_right_logical(v, 14)
                dst_b[j][pl.ds(k * LANES, LANES)] = lax.bitwise_and(v, 16383)

        def start_gather(j):
            pltpu.async_copy(table_hbm.at[src_b[j]], rows_b[j], gsem[j])

        def wait_gather(j):
            pltpu.make_async_copy(table_hbm.at[src_b[j]], rows_b[j], gsem[j]).wait()

        def start_scatter(j):
            pltpu.async_copy(rows_b[j], acc_sh.at[dst_b[j]], ssem[j], add=True)
            if with_deg:
                pltpu.async_copy(ones_v, deg_sh.at[dst_b[j]], dsem, add=True)

        def wait_scatter(j):
            pltpu.make_async_copy(rows_b[j], acc_sh.at[dst_b[j]], ssem[j]).wait()
            if with_deg:
                pltpu.make_async_copy(ones_v, deg_sh.at[dst_b[j]], dsem).wait()

        def step(c, j, wait_prev=True, issue_next=True):
            jp = (j + 2) % 3
            wait_gather(j)
            start_scatter(j)
            if wait_prev:
                wait_scatter(jp)
            if issue_next:
                unpack(c + 2, jp)
                start_gather(jp)

        unpack(0, 0)
        start_gather(0)
        unpack(1, 1)
        start_gather(1)

        zero16 = jnp.zeros((LANES,), jnp.float32)
        one16 = jnp.ones((LANES,), jnp.float32)
        for k in range(CHUNK // LANES):
            ones_v[pl.ds(k * LANES, LANES)] = one16

        @pl.loop(0, CHUNK)
        def _(i):
            for k in range(d // LANES):
                rows_b[2][i, pl.ds(k * LANES, LANES)] = zero16

        for k in range(CHUNK // LANES):
            zer_v[pl.ds(k * LANES, LANES)] = zero16

        base = sid * rows_per_tile
        for k in range(nz):
            pltpu.sync_copy(rows_b[2], acc_sh.at[pl.ds(base + k * CHUNK, CHUNK)])
        if with_deg:
            for k in range(nz):
                pltpu.sync_copy(zer_v, deg_sh.at[pl.ds(base + k * CHUNK, CHUNK)])
        plsc.subcore_barrier()

        step(0, 0, wait_prev=False)
        step(1, 1)
        step(2, 2)

        @pl.loop(0, (n_chunks - 5) // 3)
        def _(p):
            c0 = 3 * p + 3
            step(c0, 0)
            step(c0 + 1, 1)
            step(c0 + 2, 2)

        step(n_chunks - 2, 0, issue_next=False)
        step(n_chunks - 1, 1, issue_next=False)
        wait_scatter(1)

        plsc.subcore_barrier()

        pltpu.sync_copy(acc_sh.at[pl.ds(base, rows_per_tile)],
                        out_hbm.at[cid, pl.ds(base, rows_per_tile)])
        if with_deg:
            pltpu.sync_copy(deg_sh.at[pl.ds(base, rows_per_tile)],
                            deg_hbm.at[cid, pl.ds(base, rows_per_tile)])

    params = None
    if d % 128 != 0:
        params = pltpu.CompilerParams(use_tc_tiling_on_sc=False)
    return pl.kernel(body, out_type=out_type, mesh=mesh, scratch_types=scratch,
                     compiler_params=params)


def _tc_pack_body(eb, ob):
    v = eb[...]
    ob[...] = jnp.bitwise_or(jnp.left_shift(v[0:1], 14), v[1:2])


def _tc_layer0_body(xb, ab, db, ws0, wn0, b0b, ws1, wn1, os1, on1):
    rdeg = 1.0 / jnp.maximum(db[0] + db[1], 1.0)
    hn = (ab[0] + ab[1]) * rdeg
    h = xb[...] @ ws0[...] + hn @ wn0[...] + b0b[...]
    h = jnp.maximum(h, 0.0)
    os1[...] = h @ ws1[...]
    on1[...] = h @ wn1[...]


def _tc_final_body(sb, gb, db, b1b, ob):
    rdeg = 1.0 / jnp.maximum(db[0] + db[1], 1.0)
    ob[...] = sb[...] + (gb[0] + gb[1]) * rdeg + b1b[...]


def kernel(x, edge_index, W_self0, W_neigh0, b0, W_self1, W_neigh1, b1):
    n, d_in = x.shape
    e = edge_index.shape[1]
    d_hid = W_self0.shape[1]
    d_out = W_self1.shape[1]
    assert e % NW == 0
    epw = e // NW
    n_chunks = -(-epw // CHUNK)
    if n_chunks % 2 == 0:
        n_chunks += 1
    epw_pad = n_chunks * CHUNK
    n_pad = ((n + NS * CHUNK - 1) // (NS * CHUNK)) * (NS * CHUNK)

    assert n <= (1 << 14)
    be = 32000
    assert e % be == 0
    pk_flat = pl.pallas_call(
        _tc_pack_body,
        grid=(e // be,),
        in_specs=[pl.BlockSpec((2, be), lambda i: (0, i))],
        out_specs=pl.BlockSpec((1, be), lambda i: (0, i)),
        out_shape=jax.ShapeDtypeStruct((1, e), jnp.int32),
    )(edge_index)
    pk2 = pk_flat.reshape(NW, epw)
    if epw_pad != epw:
        pk2 = jnp.pad(pk2, ((0, 0), (0, epw_pad - epw)), constant_values=n)
    pk3 = pk2.reshape(NW, n_chunks, CHUNK)

    agg0_fn = _make_sc_agg(n, n_pad, d_in, n_chunks, True)
    acc0, degp = agg0_fn(x, pk3)
    degp3 = degp.reshape(NC, n_pad, 1)

    bm = 2000
    grid = (n // bm,)
    hs1, hn1 = pl.pallas_call(
        _tc_layer0_body,
        grid=grid,
        in_specs=[
            pl.BlockSpec((bm, d_in), lambda i: (i, 0)),
            pl.BlockSpec((NC, bm, d_in), lambda i: (0, i, 0)),
            pl.BlockSpec((NC, bm, 1), lambda i: (0, i, 0)),
            pl.BlockSpec((d_in, d_hid), lambda i: (0, 0)),
            pl.BlockSpec((d_in, d_hid), lambda i: (0, 0)),
            pl.BlockSpec((1, d_hid), lambda i: (0, 0)),
            pl.BlockSpec((d_hid, d_out), lambda i: (0, 0)),
            pl.BlockSpec((d_hid, d_out), lambda i: (0, 0)),
        ],
        out_specs=[
            pl.BlockSpec((bm, d_out), lambda i: (i, 0)),
            pl.BlockSpec((bm, d_out), lambda i: (i, 0)),
        ],
        out_shape=[
            jax.ShapeDtypeStruct((n, d_out), jnp.float32),
            jax.ShapeDtypeStruct((n, d_out), jnp.float32),
        ],
    )(x, acc0, degp3, W_self0, W_neigh0,
      b0.reshape(1, d_hid), W_self1, W_neigh1)

    agg1_fn = _make_sc_agg(n, n_pad, d_out, n_chunks, False)
    (acc1,) = agg1_fn(hn1, pk3)

    out = pl.pallas_call(
        _tc_final_body,
        grid=grid,
        in_specs=[
            pl.BlockSpec((bm, d_out), lambda i: (i, 0)),
            pl.BlockSpec((NC, bm, d_out), lambda i: (0, i, 0)),
            pl.BlockSpec((NC, bm, 1), lambda i: (0, i, 0)),
            pl.BlockSpec((1, d_out), lambda i: (0, 0)),
        ],
        out_specs=pl.BlockSpec((bm, d_out), lambda i: (i, 0)),
        out_shape=jax.ShapeDtypeStruct((n, d_out), jnp.float32),
    )(hs1, acc1, degp3, b1.reshape(1, d_out))

    return out


# speedup vs baseline: 1.0242x; 1.0242x over previous
import functools

import jax
import jax.numpy as jnp
from jax import lax
from jax.experimental import pallas as pl
from jax.experimental.pallas import tpu as pltpu
from jax.experimental.pallas import tpu_sc as plsc

NC = 2
NS = 16
NW = NC * NS
LANES = 16
CHUNK = 80


@functools.lru_cache(maxsize=None)
def _make_sc_agg(n_nodes, n_pad, d, n_chunks, with_deg):
    rows_per_tile = n_pad // NS
    assert rows_per_tile % CHUNK == 0
    nz = rows_per_tile // CHUNK

    mesh = plsc.VectorSubcoreMesh(core_axis_name="c", subcore_axis_name="s")

    out_type = [jax.ShapeDtypeStruct((NC, n_pad, d), jnp.float32)]
    if with_deg:
        out_type.append(jax.ShapeDtypeStruct((NC, n_pad), jnp.float32))

    assert n_chunks >= 5 and (n_chunks - 5) % 3 == 0

    scratch = [
        pltpu.VMEM((n_chunks, CHUNK), jnp.int32),
        [pltpu.VMEM((CHUNK,), jnp.int32)] * 3,
        [pltpu.VMEM((CHUNK,), jnp.int32)] * 3,
        [pltpu.VMEM((CHUNK, d), jnp.float32)] * 3,
        pltpu.VMEM((CHUNK,), jnp.float32),
        pltpu.VMEM((CHUNK,), jnp.float32),
        pltpu.VMEM_SHARED((n_pad, d), jnp.float32),
        pltpu.VMEM_SHARED((n_pad,), jnp.float32),
        [pltpu.SemaphoreType.DMA] * 3,
        [pltpu.SemaphoreType.DMA] * 3,
        pltpu.SemaphoreType.DMA,
    ]

    def body(table_hbm, pk_hbm, *refs):
        if with_deg:
            out_hbm, deg_hbm = refs[0], refs[1]
            rest = refs[2:]
        else:
            out_hbm = refs[0]
            rest = refs[1:]
        (pk_v, src_b, dst_b, rows_b, ones_v, zer_v,
         acc_sh, deg_sh, gsem, ssem, dsem) = rest

        cid = lax.axis_index("c")
        sid = lax.axis_index("s")
        wid = sid * NC + cid

        pltpu.sync_copy(pk_hbm.at[wid], pk_v)

        def unpack(c, j):
            for k in range(CHUNK // LANES):
                v = pk_v[c, pl.ds(k * LANES, LANES)]
                src_b[j][pl.ds(k * LANES, LANES)] = lax.shift_right_logical(v, 14)
                dst_b[j][pl.ds(k * LANES, LANES)] = lax.bitwise_and(v, 16383)

        def start_gather(j):
            pltpu.async_copy(table_hbm.at[src_b[j]], rows_b[j], gsem[j])

        def wait_gather(j):
            pltpu.make_async_copy(table_hbm.at[src_b[j]], rows_b[j], gsem[j]).wait()

        def start_scatter(j):
            pltpu.async_copy(rows_b[j], acc_sh.at[dst_b[j]], ssem[j], add=True)
            if with_deg:
                pltpu.async_copy(ones_v, deg_sh.at[dst_b[j]], dsem, add=True)

        def wait_scatter(j):
            pltpu.make_async_copy(rows_b[j], acc_sh.at[dst_b[j]], ssem[j]).wait()
            if with_deg:
                pltpu.make_async_copy(ones_v, deg_sh.at[dst_b[j]], dsem).wait()

        def step(c, j, wait_prev=True, issue_next=True):
            jp = (j + 2) % 3
            wait_gather(j)
            start_scatter(j)
            if wait_prev:
                wait_scatter(jp)
            if issue_next:
                unpack(c + 2, jp)
                start_gather(jp)

        unpack(0, 0)
        start_gather(0)
        unpack(1, 1)
        start_gather(1)

        zero16 = jnp.zeros((LANES,), jnp.float32)
        one16 = jnp.ones((LANES,), jnp.float32)
        for k in range(CHUNK // LANES):
            ones_v[pl.ds(k * LANES, LANES)] = one16

        @pl.loop(0, CHUNK)
        def _(i):
            for k in range(d // LANES):
                rows_b[2][i, pl.ds(k * LANES, LANES)] = zero16

        for k in range(CHUNK // LANES):
            zer_v[pl.ds(k * LANES, LANES)] = zero16

        base = sid * rows_per_tile
        for k in range(nz):
            pltpu.sync_copy(rows_b[2], acc_sh.at[pl.ds(base + k * CHUNK, CHUNK)])
        if with_deg:
            for k in range(nz):
                pltpu.sync_copy(zer_v, deg_sh.at[pl.ds(base + k * CHUNK, CHUNK)])
        plsc.subcore_barrier()

        step(0, 0, wait_prev=False)
        step(1, 1)
        step(2, 2)

        @pl.loop(0, (n_chunks - 5) // 3)
        def _(p):
            c0 = 3 * p + 3
            step(c0, 0)
            step(c0 + 1, 1)
            step(c0 + 2, 2)

        step(n_chunks - 2, 0, issue_next=False)
        step(n_chunks - 1, 1, issue_next=False)
        wait_scatter(1)

        plsc.subcore_barrier()

        pltpu.sync_copy(acc_sh.at[pl.ds(base, rows_per_tile)],
                        out_hbm.at[cid, pl.ds(base, rows_per_tile)])
        if with_deg:
            pltpu.sync_copy(deg_sh.at[pl.ds(base, rows_per_tile)],
                            deg_hbm.at[cid, pl.ds(base, rows_per_tile)])

    params = None
    if d % 128 != 0:
        params = pltpu.CompilerParams(use_tc_tiling_on_sc=False)
    return pl.kernel(body, out_type=out_type, mesh=mesh, scratch_types=scratch,
                     compiler_params=params)


def _tc_layer0_body(xb, ab, db, ws0, wn0, b0b, ws1, wn1, os1, on1, orb):
    rdeg = 1.0 / jnp.maximum(db[:, 0:1] + db[:, 1:2], 1.0)
    orb[...] = rdeg
    hn = (ab[0] + ab[1]) * rdeg
    h = xb[...] @ ws0[...] + hn @ wn0[...] + b0b[...]
    h = jnp.maximum(h, 0.0)
    os1[...] = h @ ws1[...]
    on1[...] = h @ wn1[...]


def _tc_final_body(sb, gb, rb, b1b, ob):
    ob[...] = sb[...] + (gb[0] + gb[1]) * rb[...] + b1b[...]


def kernel(x, edge_index, W_self0, W_neigh0, b0, W_self1, W_neigh1, b1):
    n, d_in = x.shape
    e = edge_index.shape[1]
    d_hid = W_self0.shape[1]
    d_out = W_self1.shape[1]
    assert e % NW == 0
    epw = e // NW
    n_chunks = -(-epw // CHUNK)
    if n_chunks % 2 == 0:
        n_chunks += 1
    epw_pad = n_chunks * CHUNK
    n_pad = ((n + NS * CHUNK - 1) // (NS * CHUNK)) * (NS * CHUNK)

    assert n <= (1 << 14)
    pk2 = jnp.bitwise_or(
        jnp.left_shift(edge_index[0], 14), edge_index[1]
    ).reshape(NW, epw)
    if epw_pad != epw:
        pk2 = jnp.pad(pk2, ((0, 0), (0, epw_pad - epw)), constant_values=n)
    pk3 = pk2.reshape(NW, n_chunks, CHUNK)

    agg0_fn = _make_sc_agg(n, n_pad, d_in, n_chunks, True)
    acc0, degp = agg0_fn(x, pk3)
    degt = degp.T

    bm = 2000
    grid = (n // bm,)
    hs1, hn1, rdeg = pl.pallas_call(
        _tc_layer0_body,
        grid=grid,
        in_specs=[
            pl.BlockSpec((bm, d_in), lambda i: (i, 0)),
            pl.BlockSpec((NC, bm, d_in), lambda i: (0, i, 0)),
            pl.BlockSpec((bm, NC), lambda i: (i, 0)),
            pl.BlockSpec((d_in, d_hid), lambda i: (0, 0)),
            pl.BlockSpec((d_in, d_hid), lambda i: (0, 0)),
            pl.BlockSpec((1, d_hid), lambda i: (0, 0)),
            pl.BlockSpec((d_hid, d_out), lambda i: (0, 0)),
            pl.BlockSpec((d_hid, d_out), lambda i: (0, 0)),
        ],
        out_specs=[
            pl.BlockSpec((bm, d_out), lambda i: (i, 0)),
            pl.BlockSpec((bm, d_out), lambda i: (i, 0)),
            pl.BlockSpec((bm, 1), lambda i: (i, 0)),
        ],
        out_shape=[
            jax.ShapeDtypeStruct((n, d_out), jnp.float32),
            jax.ShapeDtypeStruct((n, d_out), jnp.float32),
            jax.ShapeDtypeStruct((n, 1), jnp.float32),
        ],
    )(x, acc0, degt, W_self0, W_neigh0,
      b0.reshape(1, d_hid), W_self1, W_neigh1)

    agg1_fn = _make_sc_agg(n, n_pad, d_out, n_chunks, False)
    (acc1,) = agg1_fn(hn1, pk3)

    out = pl.pallas_call(
        _tc_final_body,
        grid=grid,
        in_specs=[
            pl.BlockSpec((bm, d_out), lambda i: (i, 0)),
            pl.BlockSpec((NC, bm, d_out), lambda i: (0, i, 0)),
            pl.BlockSpec((bm, 1), lambda i: (i, 0)),
            pl.BlockSpec((1, d_out), lambda i: (0, 0)),
        ],
        out_specs=pl.BlockSpec((bm, d_out), lambda i: (i, 0)),
        out_shape=jax.ShapeDtypeStruct((n, d_out), jnp.float32),
    )(hs1, acc1, rdeg, b1.reshape(1, d_out))

    return out
